# baseline (speedup 1.0000x reference)
import jax
import jax.numpy as jnp
from jax.experimental import pallas as pl
from jax.experimental.pallas import tpu as pltpu

COMMITMENT_COST = 0.25
EPS = 1e-7


def _vqvae_body(x_ref, W1_ref, b1_ref, W2_ref, b2_ref, cb_ref, Wd1_ref,
                bd1_ref, Wd2_ref, bd2_ref, loss_ref, recon_ref, idx_ref,
                acc_ref, *, n_embed, code_dim, split, batch, z_dim):
    step = pl.program_id(0)
    nsteps = pl.num_programs(0)

    x = x_ref[...]
    h = jnp.maximum(
        jnp.dot(x, W1_ref[...], preferred_element_type=jnp.float32)
        + b1_ref[...], 0.0)
    z = (jnp.dot(h, W2_ref[...], preferred_element_type=jnp.float32)
         + b2_ref[...])

    cb = cb_ref[...]
    csq = jnp.sum(cb * cb, axis=1)[None, :]

    quant_parts = []
    idx_parts = []
    for s in range(split):
        zs = z[:, s * code_dim:(s + 1) * code_dim]
        zsq = jnp.sum(zs * zs, axis=1, keepdims=True)
        cross = jnp.dot(zs, cb.T, preferred_element_type=jnp.float32)
        d = (zsq - 2.0 * cross) + csq
        mind = jnp.min(d, axis=1, keepdims=True)
        cols = jax.lax.broadcasted_iota(jnp.int32, d.shape, 1)
        idx_s = jnp.min(jnp.where(d == mind, cols, n_embed), axis=1,
                        keepdims=True)
        onehot = (cols == idx_s).astype(jnp.float32)
        q_s = jnp.dot(onehot, cb, preferred_element_type=jnp.float32)
        quant_parts.append(q_s)
        idx_parts.append(idx_s)

    quant = jnp.concatenate(quant_parts, axis=1)
    idx_ref[...] = jnp.concatenate(idx_parts, axis=1)

    diff = quant - z
    commit_part = jnp.sum(diff * diff)

    h2 = jnp.maximum(
        jnp.dot(quant.astype(jnp.bfloat16), Wd1_ref[...],
                preferred_element_type=jnp.float32)
        + bd1_ref[...], 0.0)
    logits = (jnp.dot(h2.astype(jnp.bfloat16), Wd2_ref[...],
                      preferred_element_type=jnp.float32)
              + bd2_ref[...])
    recon = jax.nn.sigmoid(logits)
    recon_ref[...] = recon

    rc = jnp.clip(recon, EPS, 1.0 - EPS)
    ce_part = jnp.sum(x * jnp.log(rc) + (1.0 - x) * jnp.log(1.0 - rc))

    @pl.when(step == 0)
    def _init():
        acc_ref[0] = ce_part
        acc_ref[1] = commit_part

    @pl.when(step != 0)
    def _accum():
        acc_ref[0] = acc_ref[0] + ce_part
        acc_ref[1] = acc_ref[1] + commit_part

    @pl.when(step == nsteps - 1)
    def _finish():
        loss_ref[0, 0] = (-(acc_ref[0] / batch)
                          + COMMITMENT_COST * (acc_ref[1] / (batch * z_dim)))


def kernel(x, W_enc1, b_enc1, W_enc2, b_enc2, codebook, W_dec1, b_dec1,
           W_dec2, b_dec2):
    B, x_dim = x.shape
    h_dim = W_enc1.shape[1]
    z_dim = W_enc2.shape[1]
    n_embed, code_dim = codebook.shape
    split = z_dim // code_dim

    tile_b = 512
    grid = (B // tile_b,)

    def body(*refs):
        _vqvae_body(*refs, n_embed=n_embed, code_dim=code_dim, split=split,
                    batch=B, z_dim=z_dim)

    full = lambda shape: pl.BlockSpec(shape, lambda i: (0,) * len(shape))

    out = pl.pallas_call(
        body,
        grid=grid,
        in_specs=[
            pl.BlockSpec((tile_b, x_dim), lambda i: (i, 0)),
            full((x_dim, h_dim)),
            full((1, h_dim)),
            full((h_dim, z_dim)),
            full((1, z_dim)),
            full((n_embed, code_dim)),
            full((z_dim, h_dim)),
            full((1, h_dim)),
            full((h_dim, x_dim)),
            full((1, x_dim)),
        ],
        out_specs=[
            pl.BlockSpec(memory_space=pltpu.SMEM),
            pl.BlockSpec((tile_b, x_dim), lambda i: (i, 0)),
            pl.BlockSpec((tile_b, split), lambda i: (i, 0)),
        ],
        out_shape=[
            jax.ShapeDtypeStruct((1, 1), jnp.float32),
            jax.ShapeDtypeStruct((B, x_dim), jnp.float32),
            jax.ShapeDtypeStruct((B, split), jnp.int32),
        ],
        scratch_shapes=[pltpu.SMEM((2,), jnp.float32)],
        compiler_params=pltpu.CompilerParams(
            dimension_semantics=("arbitrary",)),
    )(x, W_enc1, b_enc1.reshape(1, h_dim), W_enc2, b_enc2.reshape(1, z_dim),
      codebook, W_dec1.astype(jnp.bfloat16), b_dec1.reshape(1, h_dim),
      W_dec2.astype(jnp.bfloat16), b_dec2.reshape(1, x_dim))

    loss, recon, idx = out
    return (loss[0, 0], recon, idx)


# speedup vs baseline: 1.4098x; 1.4098x over previous
import jax
import jax.numpy as jnp
from jax.experimental import pallas as pl
from jax.experimental.pallas import tpu as pltpu

COMMITMENT_COST = 0.25
EPS = 1e-7
SEG = 512
BIG = 1e30


def _vqvae_body(x_ref, W1_ref, b1_ref, W2_ref, b2_ref, cbd_ref, bias_ref,
                cbg_ref, Wd1_ref, bd1_ref, Wd2_ref, bd2_ref, loss_ref,
                recon_ref, idx_ref, acc_ref, *, split, batch, z_dim,
                code_dim):
    step = pl.program_id(0)
    nsteps = pl.num_programs(0)

    x = x_ref[...]
    h = jnp.maximum(
        jnp.dot(x, W1_ref[...], preferred_element_type=jnp.float32)
        + b1_ref[...], 0.0)
    z = (jnp.dot(h, W2_ref[...], preferred_element_type=jnp.float32)
         + b2_ref[...])

    tile_b = z.shape[0]
    score = (jnp.dot(z, cbd_ref[...], preferred_element_type=jnp.float32)
             + bias_ref[...])

    colsf = jax.lax.broadcasted_iota(jnp.int32, (tile_b, SEG), 1).astype(
        jnp.float32)
    oh_parts = []
    idx_parts = []
    for s in range(split):
        sc = score[:, s * SEG:(s + 1) * SEG]
        mins = jnp.min(sc, axis=1, keepdims=True)
        idxf = jnp.min(jnp.where(sc == mins, colsf, float(SEG)), axis=1,
                       keepdims=True)
        oh_parts.append((colsf == idxf).astype(jnp.float32))
        idx_parts.append(idxf)

    idx_ref[...] = jnp.concatenate(idx_parts, axis=1).astype(jnp.int32)
    onehot = jnp.concatenate(oh_parts, axis=1)
    quant = jnp.dot(onehot, cbg_ref[...], preferred_element_type=jnp.float32)

    diff = quant - z
    commit_part = jnp.sum(diff * diff)

    h2 = jnp.maximum(
        jnp.dot(quant.astype(jnp.bfloat16), Wd1_ref[...],
                preferred_element_type=jnp.float32)
        + bd1_ref[...], 0.0)
    logits = (jnp.dot(h2.astype(jnp.bfloat16), Wd2_ref[...],
                      preferred_element_type=jnp.float32)
              + bd2_ref[...])
    recon = jax.nn.sigmoid(logits)
    recon_ref[...] = recon

    rc = jnp.clip(recon, EPS, 1.0 - EPS)
    ce_part = jnp.sum(x * jnp.log(rc) + (1.0 - x) * jnp.log(1.0 - rc))

    @pl.when(step == 0)
    def _init():
        acc_ref[0] = ce_part
        acc_ref[1] = commit_part

    @pl.when(step != 0)
    def _accum():
        acc_ref[0] = acc_ref[0] + ce_part
        acc_ref[1] = acc_ref[1] + commit_part

    @pl.when(step == nsteps - 1)
    def _finish():
        loss_ref[0, 0] = (-(acc_ref[0] / batch)
                          + COMMITMENT_COST * (acc_ref[1] / (batch * z_dim)))


def kernel(x, W_enc1, b_enc1, W_enc2, b_enc2, codebook, W_dec1, b_dec1,
           W_dec2, b_dec2):
    B, x_dim = x.shape
    h_dim = W_enc1.shape[1]
    z_dim = W_enc2.shape[1]
    n_embed, code_dim = codebook.shape
    split = z_dim // code_dim

    pad = SEG - n_embed
    cbd_seg = jnp.pad(-2.0 * codebook.T, ((0, 0), (0, pad)))
    cbg_seg = jnp.pad(codebook, ((0, pad), (0, 0)))
    eye = jnp.eye(split, dtype=codebook.dtype)
    cbd = jnp.einsum("st,ck->sctk", eye, cbd_seg).reshape(
        split * code_dim, split * SEG)
    cbg = jnp.einsum("st,kc->sktc", eye, cbg_seg).reshape(
        split * SEG, split * code_dim)
    csq = jnp.sum(codebook * codebook, axis=1)
    bias = jnp.tile(jnp.pad(csq, (0, pad), constant_values=BIG),
                    split).reshape(1, split * SEG)

    tile_b = 512
    grid = (B // tile_b,)

    def body(*refs):
        _vqvae_body(*refs, split=split, batch=B, z_dim=z_dim,
                    code_dim=code_dim)

    full = lambda shape: pl.BlockSpec(shape, lambda i: (0,) * len(shape))

    out = pl.pallas_call(
        body,
        grid=grid,
        in_specs=[
            pl.BlockSpec((tile_b, x_dim), lambda i: (i, 0)),
            full((x_dim, h_dim)),
            full((1, h_dim)),
            full((h_dim, z_dim)),
            full((1, z_dim)),
            full((z_dim, split * SEG)),
            full((1, split * SEG)),
            full((split * SEG, z_dim)),
            full((z_dim, h_dim)),
            full((1, h_dim)),
            full((h_dim, x_dim)),
            full((1, x_dim)),
        ],
        out_specs=[
            pl.BlockSpec(memory_space=pltpu.SMEM),
            pl.BlockSpec((tile_b, x_dim), lambda i: (i, 0)),
            pl.BlockSpec((tile_b, split), lambda i: (i, 0)),
        ],
        out_shape=[
            jax.ShapeDtypeStruct((1, 1), jnp.float32),
            jax.ShapeDtypeStruct((B, x_dim), jnp.float32),
            jax.ShapeDtypeStruct((B, split), jnp.int32),
        ],
        scratch_shapes=[pltpu.SMEM((2,), jnp.float32)],
        compiler_params=pltpu.CompilerParams(
            dimension_semantics=("arbitrary",)),
    )(x, W_enc1, b_enc1.reshape(1, h_dim), W_enc2, b_enc2.reshape(1, z_dim),
      cbd, bias, cbg, W_dec1.astype(jnp.bfloat16), b_dec1.reshape(1, h_dim),
      W_dec2.astype(jnp.bfloat16), b_dec2.reshape(1, x_dim))

    loss, recon, idx = out
    return (loss[0, 0], recon, idx)
